# baseline (speedup 1.0000x reference)
import jax
import jax.numpy as jnp
from jax.experimental import pallas as pl
from jax.experimental.pallas import tpu as pltpu

_NSLICES = 8


def _copy_body(src_ref, dst_ref, *sems):
    copies = []
    dim = src_ref.shape[0]
    rows = dim // _NSLICES
    for i in range(_NSLICES):
        c = pltpu.make_async_copy(
            src_ref.at[pl.ds(i * rows, rows), :],
            dst_ref.at[pl.ds(i * rows, rows), :],
            sems[i],
        )
        c.start()
        copies.append(c)
    for c in copies:
        c.wait()


def _bank_snapshot(bank):
    return pl.pallas_call(
        _copy_body,
        in_specs=[pl.BlockSpec(memory_space=pl.ANY)],
        out_specs=pl.BlockSpec(memory_space=pl.ANY),
        out_shape=jax.ShapeDtypeStruct(bank.shape, bank.dtype),
        scratch_shapes=[pltpu.SemaphoreType.DMA] * _NSLICES,
    )(bank)


def kernel(output, bank):
    return (output, _bank_snapshot(bank))


# speedup vs baseline: 11.3967x; 11.3967x over previous
import functools

import jax
import jax.numpy as jnp
from jax import lax
from jax.experimental import pallas as pl
from jax.experimental.pallas import tpu as pltpu
from jax.experimental.pallas import tpu_sc as plsc

_NWORKERS = 32
_CHUNK = 16384
_NBUF = 4


def _sc_copy_body(nchunk, src, dst, buf, *sems):
    sem_in = sems[:_NBUF]
    sem_out = sems[_NBUF:]
    per_w = nchunk * _CHUNK
    wid = lax.axis_index("s") * 2 + lax.axis_index("c")
    base = wid * per_w

    def start_load(b, off):
        pltpu.make_async_copy(
            src.at[pl.ds(off, _CHUNK)], buf.at[b], sem_in[b]).start()

    def wait_load(b):
        pltpu.make_async_copy(
            src.at[pl.ds(0, _CHUNK)], buf.at[b], sem_in[b]).wait()

    def start_store(b, off):
        pltpu.make_async_copy(
            buf.at[b], dst.at[pl.ds(off, _CHUNK)], sem_out[b]).start()

    def wait_store(b):
        pltpu.make_async_copy(
            buf.at[0], dst.at[pl.ds(0, _CHUNK)], sem_out[b]).wait()

    for b in range(_NBUF):
        start_load(b, base + b * _CHUNK)

    niter = nchunk // _NBUF

    def body(i, _):
        for b in range(_NBUF):
            wait_load(b)
            start_store(b, base + (i * _NBUF + b) * _CHUNK)
        for b in range(_NBUF):
            wait_store(b)

            @pl.when(i < niter - 1)
            def _():
                start_load(b, base + ((i + 1) * _NBUF + b) * _CHUNK)

        return 0

    lax.fori_loop(0, niter, body, 0)


def _bank_snapshot(bank):
    dim, size = bank.shape
    n = dim * size
    per_w = n // _NWORKERS
    nchunk = per_w // _CHUNK
    assert per_w % _CHUNK == 0 and nchunk % _NBUF == 0

    mesh = plsc.VectorSubcoreMesh(core_axis_name="c", subcore_axis_name="s")
    flat = bank.reshape(n)
    snap = pl.kernel(
        functools.partial(_sc_copy_body, nchunk),
        out_type=jax.ShapeDtypeStruct((n,), bank.dtype),
        mesh=mesh,
        scratch_types=(
            [pltpu.VMEM((_NBUF, _CHUNK), bank.dtype)]
            + [pltpu.SemaphoreType.DMA] * (2 * _NBUF)
        ),
    )(flat)
    return snap.reshape(dim, size)


def kernel(output, bank):
    return (output, _bank_snapshot(bank))


# speedup vs baseline: 45.2739x; 3.9725x over previous
import jax
import jax.numpy as jnp
from jax.experimental import pallas as pl


def _copy_body(src_ref, dst_ref):
    dst_ref[...] = src_ref[...]


def _bank_snapshot(bank):
    dim, size = bank.shape
    blk = 16384
    grid = size // blk
    return pl.pallas_call(
        _copy_body,
        grid=(grid,),
        in_specs=[pl.BlockSpec((dim, blk), lambda i: (0, i))],
        out_specs=pl.BlockSpec((dim, blk), lambda i: (0, i)),
        out_shape=jax.ShapeDtypeStruct(bank.shape, bank.dtype),
    )(bank)


def kernel(output, bank):
    return (output, _bank_snapshot(bank))
